# Initial kernel scaffold; baseline (speedup 1.0000x reference)
import jax
import jax.numpy as jnp
from jax.experimental import pallas as pl


def kernel(text_tokens, tags, id_tokens, class_tokens, other_tokens, coords, text_table, tag_table, id_table, class_table, other_table, fc_W, fc_b):
    raise NotImplementedError("write your pallas kernel here")


# speedup vs baseline: 4.4881x; 4.4881x over previous
import functools

import jax
import jax.numpy as jnp
from jax import lax
from jax.experimental import pallas as pl
from jax.experimental.pallas import tpu as pltpu
from jax.experimental.pallas import tpu_sc as plsc

_NC = 2
_NS = 16
_NW = _NC * _NS
_L = 16

_UD = 128
_AD = 32
_TT = 20
_OT = 10
_IT = 5
_CT = 5
_DOM = 384

_C = 8


def _tree_sum(vs):
    vs = list(vs)
    while len(vs) > 1:
        nxt = [vs[i] + vs[i + 1] for i in range(0, len(vs) - 1, 2)]
        if len(vs) % 2:
            nxt.append(vs[-1])
        vs = nxt
    return vs[0]


def _make_sc_gather(n):
    npw = n // _NW
    nch = npw // _C
    mesh = plsc.VectorSubcoreMesh(core_axis_name="c", subcore_axis_name="s")

    @functools.partial(
        pl.kernel,
        out_type=jax.ShapeDtypeStruct((n, _DOM), jnp.float32),
        mesh=mesh,
        compiler_params=pltpu.CompilerParams(use_tc_tiling_on_sc=False),
        scratch_types=[
            pltpu.VMEM((_C * _TT,), jnp.int32),
            pltpu.VMEM((_C * _OT,), jnp.int32),
            pltpu.VMEM((_C * _IT,), jnp.int32),
            pltpu.VMEM((_C * _CT,), jnp.int32),
            pltpu.VMEM((_C,), jnp.int32),
            pltpu.VMEM((_C * _TT, _UD), jnp.float32),
            pltpu.VMEM((_C * _OT, _UD), jnp.float32),
            pltpu.VMEM((_C * _IT, _AD), jnp.float32),
            pltpu.VMEM((_C * _CT, _AD), jnp.float32),
            pltpu.VMEM((_C, _AD), jnp.float32),
            pltpu.VMEM((_C * 3 + _L,), jnp.float32),
            pltpu.VMEM((_C, _DOM), jnp.float32),
            pltpu.SemaphoreType.DMA,
        ],
    )
    def sc_gather(tidx_h, tags_h, iidx_h, cidx_h, oidx_h, coor_h,
                  ttab_h, gtab_h, itab_h, ctab_h, otab_h, dom_h,
                  tix, oix, iix, cix, gix,
                  trow, orow, irow, crow, grow, coor, domv, sem):
        cid = lax.axis_index("c")
        sid = lax.axis_index("s")
        wid = sid * _NC + cid
        base0 = wid * npw
        lane = lax.iota(jnp.int32, _L)

        def chunk_body(ch, carry):
            nb = base0 + ch * _C
            pltpu.sync_copy(tidx_h.at[pl.ds(nb * _TT, _C * _TT)], tix)
            pltpu.sync_copy(oidx_h.at[pl.ds(nb * _OT, _C * _OT)], oix)
            pltpu.sync_copy(iidx_h.at[pl.ds(nb * _IT, _C * _IT)], iix)
            pltpu.sync_copy(cidx_h.at[pl.ds(nb * _CT, _C * _CT)], cix)
            pltpu.sync_copy(tags_h.at[pl.ds(nb, _C)], gix)
            pltpu.sync_copy(coor_h.at[pl.ds(nb * 3, _C * 3)],
                            coor.at[pl.ds(0, _C * 3)])
            half = _C * _TT // 2
            cps = [
                pltpu.async_copy(ttab_h.at[tix.at[pl.ds(0, half)]],
                                 trow.at[pl.ds(0, half)], sem),
                pltpu.async_copy(ttab_h.at[tix.at[pl.ds(half, half)]],
                                 trow.at[pl.ds(half, half)], sem),
                pltpu.async_copy(otab_h.at[oix], orow, sem),
                pltpu.async_copy(itab_h.at[iix], irow, sem),
                pltpu.async_copy(ctab_h.at[cix], crow, sem),
                pltpu.async_copy(gtab_h.at[gix], grow, sem),
            ]
            for cp in cps:
                cp.wait()

            zero = jnp.zeros((_L,), jnp.float32)

            def node_body(i, carry2):
                r0 = i * _TT
                for v in range(_UD // _L):
                    col = pl.ds(v * _L, _L)
                    s = _tree_sum(trow[r0 + t, col] for t in range(_TT))
                    domv[i, pl.ds(v * _L, _L)] = s * (1.0 / _TT)
                for v in range(_AD // _L):
                    domv[i, pl.ds(_UD + v * _L, _L)] = grow[i, pl.ds(v * _L, _L)]
                r0 = i * _IT
                for v in range(_AD // _L):
                    col = pl.ds(v * _L, _L)
                    s = _tree_sum(irow[r0 + t, col] for t in range(_IT))
                    domv[i, pl.ds(_UD + _AD + v * _L, _L)] = s * (1.0 / _IT)
                r0 = i * _CT
                for v in range(_AD // _L):
                    col = pl.ds(v * _L, _L)
                    s = _tree_sum(crow[r0 + t, col] for t in range(_CT))
                    domv[i, pl.ds(_UD + 2 * _AD + v * _L, _L)] = s * (1.0 / _CT)
                r0 = i * _OT
                for v in range(_UD // _L):
                    col = pl.ds(v * _L, _L)
                    s = _tree_sum(orow[r0 + t, col] for t in range(_OT))
                    domv[i, pl.ds(_UD + 3 * _AD + v * _L, _L)] = s * (1.0 / _OT)
                cvec = coor[pl.ds(i * 3, _L)]
                cvec = jnp.where(lane < 3, cvec, 0.0)
                domv[i, pl.ds(2 * _UD + 3 * _AD, _L)] = cvec
                domv[i, pl.ds(2 * _UD + 3 * _AD + _L, _L)] = zero
                return carry2

            lax.fori_loop(0, _C, node_body, 0)
            pltpu.sync_copy(domv, dom_h.at[pl.ds(nb, _C)])
            return carry

        lax.fori_loop(0, nch, chunk_body, 0)

    return sc_gather


def _mm_body(x_ref, w_ref, b_ref, o_ref):
    o_ref[...] = jnp.dot(x_ref[...], w_ref[...],
                         preferred_element_type=jnp.float32) + b_ref[...]


def _tc_matmul(dom, w_pad, b):
    n = dom.shape[0]
    bn = 1024
    return pl.pallas_call(
        _mm_body,
        grid=(n // bn,),
        in_specs=[
            pl.BlockSpec((bn, _DOM), lambda i: (i, 0)),
            pl.BlockSpec((_DOM, 256), lambda i: (0, 0)),
            pl.BlockSpec((1, 256), lambda i: (0, 0)),
        ],
        out_specs=pl.BlockSpec((bn, 256), lambda i: (i, 0)),
        out_shape=jax.ShapeDtypeStruct((n, 256), jnp.float32),
    )(dom, w_pad, b)


def kernel(text_tokens, tags, id_tokens, class_tokens, other_tokens, coords,
           text_table, tag_table, id_table, class_table, other_table,
           fc_W, fc_b):
    n = text_tokens.shape[0]
    sc_gather = _make_sc_gather(n)
    dom = sc_gather(
        text_tokens.reshape(-1), tags.reshape(-1), id_tokens.reshape(-1),
        class_tokens.reshape(-1), other_tokens.reshape(-1), coords.reshape(-1),
        text_table, tag_table, id_table, class_table, other_table)
    in_dim = fc_W.shape[0]
    w_pad = jnp.zeros((_DOM, fc_W.shape[1]), fc_W.dtype).at[:in_dim].set(fc_W)
    return _tc_matmul(dom, w_pad, fc_b.reshape(1, -1))


# speedup vs baseline: 8.6428x; 1.9257x over previous
import functools

import jax
import jax.numpy as jnp
from jax import lax
from jax.experimental import pallas as pl
from jax.experimental.pallas import tpu as pltpu
from jax.experimental.pallas import tpu_sc as plsc

_NC = 2
_NS = 16
_NW = _NC * _NS
_L = 16

_UD = 128
_AD = 32
_TT = 20
_OT = 10
_IT = 5
_CT = 5
_DOM = 384

_C = 8
_HT = _C * _TT // 2


def _tree_sum(vs):
    vs = list(vs)
    while len(vs) > 1:
        nxt = [vs[i] + vs[i + 1] for i in range(0, len(vs) - 1, 2)]
        if len(vs) % 2:
            nxt.append(vs[-1])
        vs = nxt
    return vs[0]


def _make_sc_gather(n):
    npw = n // _NW
    nch = npw // _C
    mesh = plsc.VectorSubcoreMesh(core_axis_name="c", subcore_axis_name="s")

    @functools.partial(
        pl.kernel,
        out_type=jax.ShapeDtypeStruct((n, _DOM), jnp.float32),
        mesh=mesh,
        compiler_params=pltpu.CompilerParams(use_tc_tiling_on_sc=False),
        scratch_types=[
            pltpu.VMEM((npw * _TT,), jnp.int32),
            pltpu.VMEM((npw * _OT,), jnp.int32),
            pltpu.VMEM((npw * _IT,), jnp.int32),
            pltpu.VMEM((npw * _CT,), jnp.int32),
            pltpu.VMEM((npw,), jnp.int32),
            pltpu.VMEM((npw * 3 + _L,), jnp.float32),
            pltpu.VMEM((2, _C * _TT, _UD), jnp.float32),
            pltpu.VMEM((2, _C * _OT, _UD), jnp.float32),
            pltpu.VMEM((2, _C * _IT, _AD), jnp.float32),
            pltpu.VMEM((2, _C * _CT, _AD), jnp.float32),
            pltpu.VMEM((2, _C, _AD), jnp.float32),
            pltpu.VMEM((2, _C, _DOM), jnp.float32),
            pltpu.SemaphoreType.DMA,
            pltpu.SemaphoreType.DMA,
            pltpu.SemaphoreType.DMA,
            pltpu.SemaphoreType.DMA,
        ],
    )
    def sc_gather(tidx_h, tags_h, iidx_h, cidx_h, oidx_h, coor_h,
                  ttab_h, gtab_h, itab_h, ctab_h, otab_h, dom_h,
                  tix, oix, iix, cix, gix, coor,
                  trow, orow, irow, crow, grow, domv,
                  gsem0, gsem1, wsem0, wsem1):
        cid = lax.axis_index("c")
        sid = lax.axis_index("s")
        wid = sid * _NC + cid
        base0 = wid * npw
        lane = lax.iota(jnp.int32, _L)
        gsems = (gsem0, gsem1)
        wsems = (wsem0, wsem1)

        pltpu.sync_copy(tidx_h.at[pl.ds(base0 * _TT, npw * _TT)], tix)
        pltpu.sync_copy(oidx_h.at[pl.ds(base0 * _OT, npw * _OT)], oix)
        pltpu.sync_copy(iidx_h.at[pl.ds(base0 * _IT, npw * _IT)], iix)
        pltpu.sync_copy(cidx_h.at[pl.ds(base0 * _CT, npw * _CT)], cix)
        pltpu.sync_copy(tags_h.at[pl.ds(base0, npw)], gix)
        pltpu.sync_copy(coor_h.at[pl.ds(base0 * 3, npw * 3)],
                        coor.at[pl.ds(0, npw * 3)])

        def gather_pairs(ch, slot):
            off = ch * _C
            return [
                (ttab_h.at[tix.at[pl.ds(off * _TT, _HT)]],
                 trow.at[slot, pl.ds(0, _HT)]),
                (ttab_h.at[tix.at[pl.ds(off * _TT + _HT, _HT)]],
                 trow.at[slot, pl.ds(_HT, _HT)]),
                (otab_h.at[oix.at[pl.ds(off * _OT, _C * _OT)]],
                 orow.at[slot]),
                (itab_h.at[iix.at[pl.ds(off * _IT, _C * _IT)]],
                 irow.at[slot]),
                (ctab_h.at[cix.at[pl.ds(off * _CT, _C * _CT)]],
                 crow.at[slot]),
                (gtab_h.at[gix.at[pl.ds(off, _C)]],
                 grow.at[slot]),
            ]

        def fire(ch, slot):
            for src, dst in gather_pairs(ch, slot):
                pltpu.async_copy(src, dst, gsems[slot])

        def drain(ch, slot):
            for src, dst in gather_pairs(ch, slot):
                pltpu.make_async_copy(src, dst, gsems[slot]).wait()

        fire(0, 0)

        def outer(g, carry):
            for slot in (0, 1):
                ch = g * 2 + slot
                nb = base0 + ch * _C

                @pl.when(ch + 1 < nch)
                def _():
                    fire(ch + 1, 1 - slot)

                drain(ch, slot)

                @pl.when(ch >= 2)
                def _():
                    pltpu.make_async_copy(
                        domv.at[slot], dom_h.at[pl.ds(base0, _C)],
                        wsems[slot]).wait()

                zero = jnp.zeros((_L,), jnp.float32)

                def node_body(i, carry2):
                    r0 = i * _TT
                    for v in range(_UD // _L):
                        col = pl.ds(v * _L, _L)
                        s = _tree_sum(trow[slot, r0 + t, col]
                                      for t in range(_TT))
                        domv[slot, i, pl.ds(v * _L, _L)] = s * (1.0 / _TT)
                    for v in range(_AD // _L):
                        domv[slot, i, pl.ds(_UD + v * _L, _L)] = \
                            grow[slot, i, pl.ds(v * _L, _L)]
                    r0 = i * _IT
                    for v in range(_AD // _L):
                        col = pl.ds(v * _L, _L)
                        s = _tree_sum(irow[slot, r0 + t, col]
                                      for t in range(_IT))
                        domv[slot, i, pl.ds(_UD + _AD + v * _L, _L)] = \
                            s * (1.0 / _IT)
                    r0 = i * _CT
                    for v in range(_AD // _L):
                        col = pl.ds(v * _L, _L)
                        s = _tree_sum(crow[slot, r0 + t, col]
                                      for t in range(_CT))
                        domv[slot, i, pl.ds(_UD + 2 * _AD + v * _L, _L)] = \
                            s * (1.0 / _CT)
                    r0 = i * _OT
                    for v in range(_UD // _L):
                        col = pl.ds(v * _L, _L)
                        s = _tree_sum(orow[slot, r0 + t, col]
                                      for t in range(_OT))
                        domv[slot, i, pl.ds(_UD + 3 * _AD + v * _L, _L)] = \
                            s * (1.0 / _OT)
                    cvec = coor[pl.ds((ch * _C + i) * 3, _L)]
                    cvec = jnp.where(lane < 3, cvec, 0.0)
                    domv[slot, i, pl.ds(2 * _UD + 3 * _AD, _L)] = cvec
                    domv[slot, i, pl.ds(2 * _UD + 3 * _AD + _L, _L)] = zero
                    return carry2

                lax.fori_loop(0, _C, node_body, 0)
                pltpu.async_copy(domv.at[slot], dom_h.at[pl.ds(nb, _C)],
                                 wsems[slot])
            return carry

        lax.fori_loop(0, nch // 2, outer, 0)
        pltpu.make_async_copy(domv.at[0], dom_h.at[pl.ds(base0, _C)],
                              wsem0).wait()
        pltpu.make_async_copy(domv.at[1], dom_h.at[pl.ds(base0, _C)],
                              wsem1).wait()

    return sc_gather


def _mm_body(x_ref, w_ref, b_ref, o_ref):
    o_ref[...] = jnp.dot(x_ref[...], w_ref[...],
                         preferred_element_type=jnp.float32) + b_ref[...]


def _tc_matmul(dom, w_pad, b):
    n = dom.shape[0]
    bn = 1024
    return pl.pallas_call(
        _mm_body,
        grid=(n // bn,),
        in_specs=[
            pl.BlockSpec((bn, _DOM), lambda i: (i, 0)),
            pl.BlockSpec((_DOM, 256), lambda i: (0, 0)),
            pl.BlockSpec((1, 256), lambda i: (0, 0)),
        ],
        out_specs=pl.BlockSpec((bn, 256), lambda i: (i, 0)),
        out_shape=jax.ShapeDtypeStruct((n, 256), jnp.float32),
    )(dom, w_pad, b)


def kernel(text_tokens, tags, id_tokens, class_tokens, other_tokens, coords,
           text_table, tag_table, id_table, class_table, other_table,
           fc_W, fc_b):
    n = text_tokens.shape[0]
    sc_gather = _make_sc_gather(n)
    dom = sc_gather(
        text_tokens.reshape(-1), tags.reshape(-1), id_tokens.reshape(-1),
        class_tokens.reshape(-1), other_tokens.reshape(-1), coords.reshape(-1),
        text_table, tag_table, id_table, class_table, other_table)
    in_dim = fc_W.shape[0]
    w_pad = jnp.zeros((_DOM, fc_W.shape[1]), fc_W.dtype).at[:in_dim].set(fc_W)
    return _tc_matmul(dom, w_pad, fc_b.reshape(1, -1))


# speedup vs baseline: 12.9895x; 1.5029x over previous
import functools

import jax
import jax.numpy as jnp
from jax import lax
from jax.experimental import pallas as pl
from jax.experimental.pallas import tpu as pltpu
from jax.experimental.pallas import tpu_sc as plsc

_NC = 2
_NS = 16
_NW = _NC * _NS
_L = 16

_UD = 128
_AD = 32
_TT = 20
_OT = 10
_IT = 5
_CT = 5
_DOM = 384

_C = 32


def _make_sc_gather(n):
    npw = n // _NW
    nch = npw // _C
    mesh = plsc.VectorSubcoreMesh(core_axis_name="c", subcore_axis_name="s")

    @functools.partial(
        pl.kernel,
        out_type=jax.ShapeDtypeStruct((n, _DOM), jnp.float32),
        mesh=mesh,
        compiler_params=pltpu.CompilerParams(use_tc_tiling_on_sc=False),
        scratch_types=[
            pltpu.VMEM((_TT, nch, _C), jnp.int32),
            pltpu.VMEM((_OT, nch, _C), jnp.int32),
            pltpu.VMEM((_IT, nch, _C), jnp.int32),
            pltpu.VMEM((_CT, nch, _C), jnp.int32),
            pltpu.VMEM((nch, _C), jnp.int32),
            pltpu.VMEM((2, _C, _UD), jnp.float32),
            pltpu.VMEM((2, _C, _UD), jnp.float32),
            pltpu.VMEM((2, _C, _AD), jnp.float32),
            pltpu.VMEM((2, _C, _AD), jnp.float32),
            pltpu.VMEM((2, _C, _AD), jnp.float32),
            pltpu.VMEM((2, _C, _DOM), jnp.float32),
            pltpu.SemaphoreType.DMA,
            pltpu.SemaphoreType.DMA,
            pltpu.SemaphoreType.DMA,
            pltpu.SemaphoreType.DMA,
        ],
    )
    def sc_gather(tidx_h, tags_h, iidx_h, cidx_h, oidx_h,
                  ttab_h, gtab_h, itab_h, ctab_h, otab_h, dom_h,
                  tix, oix, iix, cix, gix,
                  ta, oa, ia, ca, ga, domv,
                  gsem0, gsem1, wsem0, wsem1):
        cid = lax.axis_index("c")
        sid = lax.axis_index("s")
        wid = sid * _NC + cid
        base0 = wid * npw
        gsems = (gsem0, gsem1)
        wsems = (wsem0, wsem1)

        pltpu.sync_copy(tidx_h.at[wid], tix)
        pltpu.sync_copy(oidx_h.at[wid], oix)
        pltpu.sync_copy(iidx_h.at[wid], iix)
        pltpu.sync_copy(cidx_h.at[wid], cix)
        pltpu.sync_copy(tags_h.at[wid], gix)

        def gather_pairs(ch, slot):
            pairs = []
            for g in range(_TT):
                pairs.append((ttab_h.at[tix.at[g, ch]], ta.at[slot], True))
            for g in range(_OT):
                pairs.append((otab_h.at[oix.at[g, ch]], oa.at[slot], True))
            for g in range(_IT):
                pairs.append((itab_h.at[iix.at[g, ch]], ia.at[slot], True))
            for g in range(_CT):
                pairs.append((ctab_h.at[cix.at[g, ch]], ca.at[slot], True))
            pairs.append((gtab_h.at[gix.at[ch]], ga.at[slot], False))
            return pairs

        def fire(ch, slot):
            for src, dst, add in gather_pairs(ch, slot):
                pltpu.async_copy(src, dst, gsems[slot], add=add)

        def drain(ch, slot):
            for src, dst, add in gather_pairs(ch, slot):
                pltpu.make_async_copy(src, dst, gsems[slot]).wait()

        zero = jnp.zeros((_L,), jnp.float32)

        def zero_accs(i, carry):
            for slot in (0, 1):
                for v in range(_UD // _L):
                    ta[slot, i, pl.ds(v * _L, _L)] = zero
                    oa[slot, i, pl.ds(v * _L, _L)] = zero
                for v in range(_AD // _L):
                    ia[slot, i, pl.ds(v * _L, _L)] = zero
                    ca[slot, i, pl.ds(v * _L, _L)] = zero
            return carry

        lax.fori_loop(0, _C, zero_accs, 0)

        fire(0, 0)

        def outer(g, carry):
            for slot in (0, 1):
                ch = g * 2 + slot
                nb = base0 + ch * _C

                @pl.when(ch + 1 < nch)
                def _():
                    fire(ch + 1, 1 - slot)

                drain(ch, slot)

                @pl.when(ch >= 2)
                def _():
                    pltpu.make_async_copy(
                        domv.at[slot], dom_h.at[pl.ds(base0, _C)],
                        wsems[slot]).wait()

                def node_body(i, carry2):
                    for v in range(_UD // _L):
                        col = pl.ds(v * _L, _L)
                        domv[slot, i, col] = ta[slot, i, col] * (1.0 / _TT)
                        ta[slot, i, col] = zero
                    for v in range(_AD // _L):
                        col = pl.ds(v * _L, _L)
                        domv[slot, i, pl.ds(_UD + v * _L, _L)] = \
                            ga[slot, i, col]
                    for v in range(_AD // _L):
                        col = pl.ds(v * _L, _L)
                        domv[slot, i, pl.ds(_UD + _AD + v * _L, _L)] = \
                            ia[slot, i, col] * (1.0 / _IT)
                        ia[slot, i, col] = zero
                    for v in range(_AD // _L):
                        col = pl.ds(v * _L, _L)
                        domv[slot, i, pl.ds(_UD + 2 * _AD + v * _L, _L)] = \
                            ca[slot, i, col] * (1.0 / _CT)
                        ca[slot, i, col] = zero
                    for v in range(_UD // _L):
                        col = pl.ds(v * _L, _L)
                        domv[slot, i, pl.ds(_UD + 3 * _AD + v * _L, _L)] = \
                            oa[slot, i, col] * (1.0 / _OT)
                        oa[slot, i, col] = zero
                    domv[slot, i, pl.ds(2 * _UD + 3 * _AD, _L)] = zero
                    domv[slot, i, pl.ds(2 * _UD + 3 * _AD + _L, _L)] = zero
                    return carry2

                lax.fori_loop(0, _C, node_body, 0)
                pltpu.async_copy(domv.at[slot], dom_h.at[pl.ds(nb, _C)],
                                 wsems[slot])
            return carry

        lax.fori_loop(0, nch // 2, outer, 0)
        pltpu.make_async_copy(domv.at[0], dom_h.at[pl.ds(base0, _C)],
                              wsem0).wait()
        pltpu.make_async_copy(domv.at[1], dom_h.at[pl.ds(base0, _C)],
                              wsem1).wait()

    return sc_gather


def _mm_body(x_ref, c_ref, w_ref, w2_ref, b_ref, o_ref):
    o_ref[...] = (jnp.dot(x_ref[...], w_ref[...],
                          preferred_element_type=jnp.float32)
                  + jnp.dot(c_ref[...], w2_ref[...],
                            preferred_element_type=jnp.float32)
                  + b_ref[...])


def _tc_matmul(dom, coords, w_pad, w_c, b):
    n = dom.shape[0]
    bn = 1024
    return pl.pallas_call(
        _mm_body,
        grid=(n // bn,),
        in_specs=[
            pl.BlockSpec((bn, _DOM), lambda i: (i, 0)),
            pl.BlockSpec((bn, 3), lambda i: (i, 0)),
            pl.BlockSpec((_DOM, 256), lambda i: (0, 0)),
            pl.BlockSpec((3, 256), lambda i: (0, 0)),
            pl.BlockSpec((1, 256), lambda i: (0, 0)),
        ],
        out_specs=pl.BlockSpec((bn, 256), lambda i: (i, 0)),
        out_shape=jax.ShapeDtypeStruct((n, 256), jnp.float32),
    )(dom, coords, w_pad, w_c, b)


def kernel(text_tokens, tags, id_tokens, class_tokens, other_tokens, coords,
           text_table, tag_table, id_table, class_table, other_table,
           fc_W, fc_b):
    n = text_tokens.shape[0]
    npw = n // _NW
    nch = npw // _C

    def tposed(tok, k):
        return (tok.reshape(_NW, nch, _C, k)
                .transpose(0, 3, 1, 2))

    sc_gather = _make_sc_gather(n)
    dom = sc_gather(
        tposed(text_tokens, _TT), tags.reshape(_NW, nch, _C),
        tposed(id_tokens, _IT), tposed(class_tokens, _CT),
        tposed(other_tokens, _OT),
        text_table, tag_table, id_table, class_table, other_table)
    in_dim = fc_W.shape[0]
    w_pad = jnp.zeros((_DOM, fc_W.shape[1]), fc_W.dtype).at[:in_dim].set(fc_W)
    w_c = fc_W[2 * _UD + 3 * _AD:in_dim]
    return _tc_matmul(dom, coords, w_pad, w_c, fc_b.reshape(1, -1))


# speedup vs baseline: 13.2112x; 1.0171x over previous
import functools

import jax
import jax.numpy as jnp
from jax import lax
from jax.experimental import pallas as pl
from jax.experimental.pallas import tpu as pltpu
from jax.experimental.pallas import tpu_sc as plsc

_NC = 2
_NS = 16
_NW = _NC * _NS
_L = 16

_UD = 128
_AD = 32
_TT = 20
_OT = 10
_IT = 5
_CT = 5
_DOM = 384

_C = 64


def _make_sc_gather(n):
    npw = n // _NW
    nch = npw // _C
    mesh = plsc.VectorSubcoreMesh(core_axis_name="c", subcore_axis_name="s")

    @functools.partial(
        pl.kernel,
        out_type=jax.ShapeDtypeStruct((n, _DOM), jnp.float32),
        mesh=mesh,
        compiler_params=pltpu.CompilerParams(use_tc_tiling_on_sc=False),
        scratch_types=[
            pltpu.VMEM((_TT, nch, _C), jnp.int32),
            pltpu.VMEM((_OT, nch, _C), jnp.int32),
            pltpu.VMEM((_IT, nch, _C), jnp.int32),
            pltpu.VMEM((_CT, nch, _C), jnp.int32),
            pltpu.VMEM((nch, _C), jnp.int32),
            pltpu.VMEM((2, _C, _UD), jnp.float32),
            pltpu.VMEM((2, _C, _UD), jnp.float32),
            pltpu.VMEM((2, _C, _AD), jnp.float32),
            pltpu.VMEM((2, _C, _AD), jnp.float32),
            pltpu.VMEM((2, _C, _AD), jnp.float32),
            pltpu.VMEM((2, _C, _DOM), jnp.float32),
            pltpu.SemaphoreType.DMA,
            pltpu.SemaphoreType.DMA,
            pltpu.SemaphoreType.DMA,
            pltpu.SemaphoreType.DMA,
        ],
    )
    def sc_gather(tidx_h, tags_h, iidx_h, cidx_h, oidx_h,
                  ttab_h, gtab_h, itab_h, ctab_h, otab_h, dom_h,
                  tix, oix, iix, cix, gix,
                  ta, oa, ia, ca, ga, domv,
                  gsem0, gsem1, wsem0, wsem1):
        cid = lax.axis_index("c")
        sid = lax.axis_index("s")
        wid = sid * _NC + cid
        base0 = wid * npw
        gsems = (gsem0, gsem1)
        wsems = (wsem0, wsem1)

        pltpu.sync_copy(tidx_h.at[wid], tix)
        pltpu.sync_copy(oidx_h.at[wid], oix)
        pltpu.sync_copy(iidx_h.at[wid], iix)
        pltpu.sync_copy(cidx_h.at[wid], cix)
        pltpu.sync_copy(tags_h.at[wid], gix)

        def gather_pairs(ch, slot):
            pairs = []
            for g in range(_TT):
                pairs.append((ttab_h.at[tix.at[g, ch]], ta.at[slot], True))
            for g in range(_OT):
                pairs.append((otab_h.at[oix.at[g, ch]], oa.at[slot], True))
            for g in range(_IT):
                pairs.append((itab_h.at[iix.at[g, ch]], ia.at[slot], True))
            for g in range(_CT):
                pairs.append((ctab_h.at[cix.at[g, ch]], ca.at[slot], True))
            pairs.append((gtab_h.at[gix.at[ch]], ga.at[slot], False))
            return pairs

        def fire(ch, slot):
            for src, dst, add in gather_pairs(ch, slot):
                pltpu.async_copy(src, dst, gsems[slot], add=add)

        def drain(ch, slot):
            for src, dst, add in gather_pairs(ch, slot):
                pltpu.make_async_copy(src, dst, gsems[slot]).wait()

        zero = jnp.zeros((_L,), jnp.float32)

        def zero_accs(i, carry):
            for slot in (0, 1):
                for v in range(_UD // _L):
                    ta[slot, i, pl.ds(v * _L, _L)] = zero
                    oa[slot, i, pl.ds(v * _L, _L)] = zero
                for v in range(_AD // _L):
                    ia[slot, i, pl.ds(v * _L, _L)] = zero
                    ca[slot, i, pl.ds(v * _L, _L)] = zero
            return carry

        lax.fori_loop(0, _C, zero_accs, 0)

        fire(0, 0)

        def outer(g, carry):
            for slot in (0, 1):
                ch = g * 2 + slot
                nb = base0 + ch * _C

                @pl.when(ch + 1 < nch)
                def _():
                    fire(ch + 1, 1 - slot)

                drain(ch, slot)

                @pl.when(ch >= 2)
                def _():
                    pltpu.make_async_copy(
                        domv.at[slot], dom_h.at[pl.ds(base0, _C)],
                        wsems[slot]).wait()

                def node_body(i, carry2):
                    for v in range(_UD // _L):
                        col = pl.ds(v * _L, _L)
                        domv[slot, i, col] = ta[slot, i, col] * (1.0 / _TT)
                        ta[slot, i, col] = zero
                    for v in range(_AD // _L):
                        col = pl.ds(v * _L, _L)
                        domv[slot, i, pl.ds(_UD + v * _L, _L)] = \
                            ga[slot, i, col]
                    for v in range(_AD // _L):
                        col = pl.ds(v * _L, _L)
                        domv[slot, i, pl.ds(_UD + _AD + v * _L, _L)] = \
                            ia[slot, i, col] * (1.0 / _IT)
                        ia[slot, i, col] = zero
                    for v in range(_AD // _L):
                        col = pl.ds(v * _L, _L)
                        domv[slot, i, pl.ds(_UD + 2 * _AD + v * _L, _L)] = \
                            ca[slot, i, col] * (1.0 / _CT)
                        ca[slot, i, col] = zero
                    for v in range(_UD // _L):
                        col = pl.ds(v * _L, _L)
                        domv[slot, i, pl.ds(_UD + 3 * _AD + v * _L, _L)] = \
                            oa[slot, i, col] * (1.0 / _OT)
                        oa[slot, i, col] = zero
                    domv[slot, i, pl.ds(2 * _UD + 3 * _AD, _L)] = zero
                    domv[slot, i, pl.ds(2 * _UD + 3 * _AD + _L, _L)] = zero
                    return carry2

                lax.fori_loop(0, _C, node_body, 0)
                pltpu.async_copy(domv.at[slot], dom_h.at[pl.ds(nb, _C)],
                                 wsems[slot])
            return carry

        lax.fori_loop(0, nch // 2, outer, 0)
        pltpu.make_async_copy(domv.at[0], dom_h.at[pl.ds(base0, _C)],
                              wsem0).wait()
        pltpu.make_async_copy(domv.at[1], dom_h.at[pl.ds(base0, _C)],
                              wsem1).wait()

    return sc_gather


def _mm_body(x_ref, c_ref, w_ref, w2_ref, b_ref, o_ref):
    o_ref[...] = (jnp.dot(x_ref[...], w_ref[...],
                          preferred_element_type=jnp.float32)
                  + jnp.dot(c_ref[...], w2_ref[...],
                            preferred_element_type=jnp.float32)
                  + b_ref[...])


def _tc_matmul(dom, coords, w_pad, w_c, b):
    n = dom.shape[0]
    bn = 1024
    return pl.pallas_call(
        _mm_body,
        grid=(n // bn,),
        in_specs=[
            pl.BlockSpec((bn, _DOM), lambda i: (i, 0)),
            pl.BlockSpec((bn, 3), lambda i: (i, 0)),
            pl.BlockSpec((_DOM, 256), lambda i: (0, 0)),
            pl.BlockSpec((3, 256), lambda i: (0, 0)),
            pl.BlockSpec((1, 256), lambda i: (0, 0)),
        ],
        out_specs=pl.BlockSpec((bn, 256), lambda i: (i, 0)),
        out_shape=jax.ShapeDtypeStruct((n, 256), jnp.float32),
    )(dom, coords, w_pad, w_c, b)


def kernel(text_tokens, tags, id_tokens, class_tokens, other_tokens, coords,
           text_table, tag_table, id_table, class_table, other_table,
           fc_W, fc_b):
    n = text_tokens.shape[0]
    npw = n // _NW
    nch = npw // _C

    def tposed(tok, k):
        return (tok.reshape(_NW, nch, _C, k)
                .transpose(0, 3, 1, 2))

    sc_gather = _make_sc_gather(n)
    dom = sc_gather(
        tposed(text_tokens, _TT), tags.reshape(_NW, nch, _C),
        tposed(id_tokens, _IT), tposed(class_tokens, _CT),
        tposed(other_tokens, _OT),
        text_table, tag_table, id_table, class_table, other_table)
    in_dim = fc_W.shape[0]
    w_pad = jnp.zeros((_DOM, fc_W.shape[1]), fc_W.dtype).at[:in_dim].set(fc_W)
    w_c = fc_W[2 * _UD + 3 * _AD:in_dim]
    return _tc_matmul(dom, coords, w_pad, w_c, fc_b.reshape(1, -1))


# speedup vs baseline: 15.2287x; 1.1527x over previous
import functools

import jax
import jax.numpy as jnp
from jax import lax
from jax.experimental import pallas as pl
from jax.experimental.pallas import tpu as pltpu
from jax.experimental.pallas import tpu_sc as plsc

_NC = 2
_NS = 16
_NW = _NC * _NS
_L = 16

_UD = 128
_AD = 32
_TT = 20
_OT = 10
_IT = 5
_CT = 5
_DOM = 384

_C = 64


def _make_sc_gather(n):
    npw = n // _NW
    nch = npw // _C
    mesh = plsc.VectorSubcoreMesh(core_axis_name="c", subcore_axis_name="s")

    @functools.partial(
        pl.kernel,
        out_type=jax.ShapeDtypeStruct((n * (_DOM // 128), 128), jnp.float32),
        mesh=mesh,
        compiler_params=pltpu.CompilerParams(use_tc_tiling_on_sc=False),
        scratch_types=[
            pltpu.VMEM((_TT, nch, _C), jnp.int32),
            pltpu.VMEM((_OT, nch, _C), jnp.int32),
            pltpu.VMEM((_IT, nch, _C), jnp.int32),
            pltpu.VMEM((_CT, nch, _C), jnp.int32),
            pltpu.VMEM((nch, _C), jnp.int32),
            pltpu.VMEM((2, _C, _UD), jnp.float32),
            pltpu.VMEM((2, _C, _UD), jnp.float32),
            pltpu.VMEM((2, _C, _AD), jnp.float32),
            pltpu.VMEM((2, _C, _AD), jnp.float32),
            pltpu.VMEM((2, _C, _AD), jnp.float32),
            pltpu.VMEM((2, _C * (_DOM // 128), 128),
                       jnp.float32),
            pltpu.SemaphoreType.DMA,
            pltpu.SemaphoreType.DMA,
            pltpu.SemaphoreType.DMA,
            pltpu.SemaphoreType.DMA,
        ],
    )
    def sc_gather(tidx_h, tags_h, iidx_h, cidx_h, oidx_h,
                  ttab_h, gtab_h, itab_h, ctab_h, otab_h, dom_h,
                  tix, oix, iix, cix, gix,
                  ta, oa, ia, ca, ga, domv,
                  gsem0, gsem1, wsem0, wsem1):
        cid = lax.axis_index("c")
        sid = lax.axis_index("s")
        wid = sid * _NC + cid
        base0 = wid * npw
        gsems = (gsem0, gsem1)
        wsems = (wsem0, wsem1)

        pltpu.sync_copy(tidx_h.at[wid], tix)
        pltpu.sync_copy(oidx_h.at[wid], oix)
        pltpu.sync_copy(iidx_h.at[wid], iix)
        pltpu.sync_copy(cidx_h.at[wid], cix)
        pltpu.sync_copy(tags_h.at[wid], gix)

        def gather_pairs(ch, slot):
            pairs = []
            for g in range(_TT):
                pairs.append((ttab_h.at[tix.at[g, ch]], ta.at[slot], True))
            for g in range(_OT):
                pairs.append((otab_h.at[oix.at[g, ch]], oa.at[slot], True))
            for g in range(_IT):
                pairs.append((itab_h.at[iix.at[g, ch]], ia.at[slot], True))
            for g in range(_CT):
                pairs.append((ctab_h.at[cix.at[g, ch]], ca.at[slot], True))
            pairs.append((gtab_h.at[gix.at[ch]], ga.at[slot], False))
            return pairs

        def fire(ch, slot):
            for src, dst, add in gather_pairs(ch, slot):
                pltpu.async_copy(src, dst, gsems[slot], add=add)

        def drain(ch, slot):
            for src, dst, add in gather_pairs(ch, slot):
                pltpu.make_async_copy(src, dst, gsems[slot]).wait()

        zero = jnp.zeros((_L,), jnp.float32)

        def zero_accs(i, carry):
            for slot in (0, 1):
                for v in range(_UD // _L):
                    ta[slot, i, pl.ds(v * _L, _L)] = zero
                    oa[slot, i, pl.ds(v * _L, _L)] = zero
                for v in range(_AD // _L):
                    ia[slot, i, pl.ds(v * _L, _L)] = zero
                    ca[slot, i, pl.ds(v * _L, _L)] = zero
            return carry

        lax.fori_loop(0, _C, zero_accs, 0)

        fire(0, 0)

        def outer(g, carry):
            for slot in (0, 1):
                ch = g * 2 + slot
                nb = base0 + ch * _C

                @pl.when(ch + 1 < nch)
                def _():
                    fire(ch + 1, 1 - slot)

                drain(ch, slot)

                @pl.when(ch >= 2)
                def _():
                    pltpu.make_async_copy(
                        domv.at[slot], dom_h.at[pl.ds(base0 * 3, _C * 3)],
                        wsems[slot]).wait()

                def node_body(i, carry2):
                    rbase = (i >> 3) * (3 * 8) + (i & 7)

                    def dom_store(col, val):
                        domv[slot, rbase + (col // 128) * 8,
                             pl.ds(col % 128, _L)] = val

                    for v in range(_UD // _L):
                        col = pl.ds(v * _L, _L)
                        dom_store(v * _L, ta[slot, i, col] * (1.0 / _TT))
                        ta[slot, i, col] = zero
                    for v in range(_AD // _L):
                        col = pl.ds(v * _L, _L)
                        dom_store(_UD + v * _L, ga[slot, i, col])
                    for v in range(_AD // _L):
                        col = pl.ds(v * _L, _L)
                        dom_store(_UD + _AD + v * _L,
                                  ia[slot, i, col] * (1.0 / _IT))
                        ia[slot, i, col] = zero
                    for v in range(_AD // _L):
                        col = pl.ds(v * _L, _L)
                        dom_store(_UD + 2 * _AD + v * _L,
                                  ca[slot, i, col] * (1.0 / _CT))
                        ca[slot, i, col] = zero
                    for v in range(_UD // _L):
                        col = pl.ds(v * _L, _L)
                        dom_store(_UD + 3 * _AD + v * _L,
                                  oa[slot, i, col] * (1.0 / _OT))
                        oa[slot, i, col] = zero
                    dom_store(2 * _UD + 3 * _AD, zero)
                    dom_store(2 * _UD + 3 * _AD + _L, zero)
                    return carry2

                lax.fori_loop(0, _C, node_body, 0)
                pltpu.async_copy(domv.at[slot],
                                 dom_h.at[pl.ds(nb * 3, _C * 3)],
                                 wsems[slot])
            return carry

        lax.fori_loop(0, nch // 2, outer, 0)
        pltpu.make_async_copy(domv.at[0], dom_h.at[pl.ds(base0 * 3, _C * 3)],
                              wsem0).wait()
        pltpu.make_async_copy(domv.at[1], dom_h.at[pl.ds(base0 * 3, _C * 3)],
                              wsem1).wait()

    return sc_gather


def _mm_body(x_ref, c_ref, w_ref, w2_ref, b_ref, o_ref):
    bn = x_ref.shape[0] // 3
    w = w_ref[...]
    x4 = x_ref[...].reshape(bn // 8, 3, 8, 128)
    acc = jnp.dot(c_ref[...], w2_ref[...], preferred_element_type=jnp.float32)
    for c in range(_DOM // 128):
        xc = x4[:, c].reshape(bn, 128)
        acc = acc + jnp.dot(xc, w[c * 128:(c + 1) * 128],
                            preferred_element_type=jnp.float32)
    o_ref[...] = acc + b_ref[...]


def _tc_matmul(dom_t, coords, w_pad, w_c, b):
    n = dom_t.shape[0] // 3
    bn = 1024
    return pl.pallas_call(
        _mm_body,
        grid=(n // bn,),
        in_specs=[
            pl.BlockSpec((bn * 3, 128), lambda i: (i, 0)),
            pl.BlockSpec((bn, 3), lambda i: (i, 0)),
            pl.BlockSpec((_DOM, 256), lambda i: (0, 0)),
            pl.BlockSpec((3, 256), lambda i: (0, 0)),
            pl.BlockSpec((1, 256), lambda i: (0, 0)),
        ],
        out_specs=pl.BlockSpec((bn, 256), lambda i: (i, 0)),
        out_shape=jax.ShapeDtypeStruct((n, 256), jnp.float32),
    )(dom_t, coords, w_pad, w_c, b)


def kernel(text_tokens, tags, id_tokens, class_tokens, other_tokens, coords,
           text_table, tag_table, id_table, class_table, other_table,
           fc_W, fc_b):
    n = text_tokens.shape[0]
    npw = n // _NW
    nch = npw // _C

    def tposed(tok, k):
        return (tok.reshape(_NW, nch, _C, k)
                .transpose(0, 3, 1, 2))

    sc_gather = _make_sc_gather(n)
    dom = sc_gather(
        tposed(text_tokens, _TT), tags.reshape(_NW, nch, _C),
        tposed(id_tokens, _IT), tposed(class_tokens, _CT),
        tposed(other_tokens, _OT),
        text_table, tag_table, id_table, class_table, other_table)
    in_dim = fc_W.shape[0]
    w_pad = jnp.zeros((_DOM, fc_W.shape[1]), fc_W.dtype).at[:in_dim].set(fc_W)
    w_c = fc_W[2 * _UD + 3 * _AD:in_dim]
    return _tc_matmul(dom, coords, w_pad, w_c, fc_b.reshape(1, -1))
